# baseline (speedup 1.0000x reference)
import jax
import jax.numpy as jnp
from jax import lax
from jax.experimental import pallas as pl
from jax.experimental.pallas import tpu as pltpu
from jax.experimental.pallas import tpu_sc as plsc

N = 10000
E = 160000
IN_CH = 256
OUT_CH = 256
HEADS = 4
EDGE_DIM = 4

NPAD = 10112
TW = 1 + EDGE_DIM
NQ = 8
QBLK = 20480
EPAD = NQ * QBLK
ECHUNK = 2048
NCH = QBLK // ECHUNK


def _a_body(x_ref, wn_ref, att_ref, a_ref):
    cols = []
    for h in range(HEADS):
        vh = jnp.sum(wn_ref[:, h * OUT_CH:(h + 1) * OUT_CH] * att_ref[h, :][None, :],
                     axis=1)
        cols.append(vh[:, None])
    v = jnp.concatenate(cols, axis=1)
    a_ref[...] = jnp.dot(x_ref[...], v, preferred_element_type=jnp.float32)


def _compute_a(x, w_node, att2):
    blk = 2000
    return pl.pallas_call(
        _a_body,
        grid=(N // blk,),
        in_specs=[
            pl.BlockSpec((blk, IN_CH), lambda i: (i, 0)),
            pl.BlockSpec((IN_CH, OUT_CH * HEADS), lambda i: (0, 0)),
            pl.BlockSpec((HEADS, OUT_CH), lambda i: (0, 0)),
        ],
        out_specs=pl.BlockSpec((blk, HEADS), lambda i: (i, 0)),
        out_shape=jax.ShapeDtypeStruct((N, HEADS), jnp.float32),
    )(x, w_node, att2)


def _b_body(ea_ref, we_ref, att_ref, b_ref):
    for h in range(HEADS):
        acc = None
        for d in range(EDGE_DIM):
            w_dh = jnp.sum(we_ref[d, h * OUT_CH:(h + 1) * OUT_CH] * att_ref[h, :])
            term = ea_ref[d:d + 1, :] * w_dh
            acc = term if acc is None else acc + term
        b_ref[h:h + 1, :] = acc


def _compute_b(ea_t, w_edge, att2):
    blk = EPAD // 4
    return pl.pallas_call(
        _b_body,
        grid=(EPAD // blk,),
        in_specs=[
            pl.BlockSpec((EDGE_DIM, blk), lambda i: (0, i)),
            pl.BlockSpec((EDGE_DIM, OUT_CH * HEADS), lambda i: (0, 0)),
            pl.BlockSpec((HEADS, OUT_CH), lambda i: (0, 0)),
        ],
        out_specs=pl.BlockSpec((HEADS, blk), lambda i: (0, i)),
        out_shape=jax.ShapeDtypeStruct((EDGE_DIM, EPAD), jnp.float32),
    )(ea_t, w_edge, att2)


def _sc_body(col_hbm, b_hbm, ea_hbm, at_hbm, out_hbm,
             a_v, col_v, b_v, ea_v, tab_v, sem_in, sem_e0, sem_e1):
    cid = lax.axis_index("c")
    sid = lax.axis_index("s")
    h = sid % HEADS
    q = cid * (NQ // 2) + sid // HEADS
    e_base = q * QBLK

    in_descs = [
        pltpu.async_copy(col_hbm.at[pl.ds(e_base, QBLK)], col_v, sem_in),
        pltpu.async_copy(at_hbm.at[pl.ds(h * NPAD, NPAD)], a_v, sem_in),
        pltpu.async_copy(b_hbm.at[pl.ds(h * EPAD + e_base, QBLK)], b_v,
                         sem_in),
    ]

    sem_e = [sem_e0, sem_e1]

    def fire(ci, buf):
        ds = []
        for d in range(EDGE_DIM):
            ds.append(pltpu.async_copy(
                ea_hbm.at[pl.ds(d * EPAD + e_base + ci * ECHUNK, ECHUNK)],
                ea_v.at[buf, d], sem_e[buf]))
        return ds

    ea_descs = [fire(0, 0), None]

    zero16 = jnp.zeros((16,), jnp.float32)

    @pl.loop(0, NPAD * TW // 16, unroll=8)
    def _zero(i):
        tab_v[pl.ds(i * 16, 16)] = zero16

    for dsc in in_descs:
        dsc.wait()

    for ci in range(NCH):
        buf = ci % 2
        if ci + 1 < NCH:
            ea_descs[1 - buf] = fire(ci + 1, 1 - buf)
        for dsc in ea_descs[buf]:
            dsc.wait()

        @pl.loop(0, ECHUNK // 16)
        def _group(g, ci=ci, buf=buf):
            e0 = ci * ECHUNK + g * 16
            col16 = col_v[pl.ds(e0, 16)]
            ah = plsc.load_gather(a_v, [col16])
            bh = b_v[pl.ds(e0, 16)]
            t = ah + bh
            t = jnp.maximum(t, 0.2 * t)
            p = jnp.exp(t)
            idx = col16 * TW
            plsc.addupdate_scatter(tab_v, [idx], p)
            for d in range(EDGE_DIM):
                ead = ea_v[buf, d, pl.ds(g * 16, 16)]
                plsc.addupdate_scatter(tab_v, [idx + (1 + d)], p * ead)

    pltpu.sync_copy(tab_v, out_hbm.at[cid, sid])


def _sc_tables(col_f, b_f, ea_f, at_f):
    mesh = plsc.VectorSubcoreMesh(core_axis_name="c", subcore_axis_name="s",
                                  num_cores=2, num_subcores=16)
    fn = pl.kernel(
        _sc_body,
        out_type=jax.ShapeDtypeStruct((2, 16, NPAD * TW), jnp.float32),
        mesh=mesh,
        compiler_params=pltpu.CompilerParams(needs_layout_passes=False,
                                             use_tc_tiling_on_sc=False),
        scratch_types=[
            pltpu.VMEM((NPAD,), jnp.float32),
            pltpu.VMEM((QBLK,), jnp.int32),
            pltpu.VMEM((QBLK,), jnp.float32),
            pltpu.VMEM((2, EDGE_DIM, ECHUNK), jnp.float32),
            pltpu.VMEM((NPAD * TW,), jnp.float32),
            pltpu.SemaphoreType.DMA,
            pltpu.SemaphoreType.DMA,
            pltpu.SemaphoreType.DMA,
        ],
    )
    return fn(col_f, b_f, ea_f, at_f)


def _out_body(x_ref, t_ref, wn_ref, we_ref, wo_ref, bo_ref, o_ref):
    x = x_ref[...]
    acc = None
    for h in range(HEADS):
        th = None
        for c in range(2):
            for k in range(NQ // 2):
                sl = t_ref[c * 16 + k * HEADS + h]
                th = sl if th is None else th + sl
        s = th[:, 0:1]
        inv = 1.0 / (s + 1e-8)
        sh = s * inv
        gh = th[:, 1:1 + EDGE_DIM] * inv
        xh = jnp.dot(x, wn_ref[:, h * OUT_CH:(h + 1) * OUT_CH],
                     preferred_element_type=jnp.float32)
        zh = jnp.dot(gh, we_ref[:, h * OUT_CH:(h + 1) * OUT_CH],
                     preferred_element_type=jnp.float32)
        term = xh * sh + zh
        contrib = jnp.dot(term, wo_ref[h * OUT_CH:(h + 1) * OUT_CH, :],
                          preferred_element_type=jnp.float32)
        acc = contrib if acc is None else acc + contrib
    o_ref[...] = acc + bo_ref[...]


def _compute_out(x, tables, w_node, w_edge, w_out, b_out2):
    blk = 400
    return pl.pallas_call(
        _out_body,
        grid=(N // blk,),
        in_specs=[
            pl.BlockSpec((blk, IN_CH), lambda i: (i, 0)),
            pl.BlockSpec((32, blk, TW), lambda i: (0, i, 0)),
            pl.BlockSpec((IN_CH, OUT_CH * HEADS), lambda i: (0, 0)),
            pl.BlockSpec((EDGE_DIM, OUT_CH * HEADS), lambda i: (0, 0)),
            pl.BlockSpec((OUT_CH * HEADS, OUT_CH), lambda i: (0, 0)),
            pl.BlockSpec((1, OUT_CH), lambda i: (0, 0)),
        ],
        out_specs=pl.BlockSpec((blk, OUT_CH), lambda i: (i, 0)),
        out_shape=jax.ShapeDtypeStruct((N, OUT_CH), jnp.float32),
    )(x, tables, w_node, w_edge, w_out, b_out2)


def kernel(x, edge_index, edge_attr, W_node, W_edge, att, W_out, b_out):
    att2 = att.reshape(HEADS, OUT_CH)
    col = edge_index[1].astype(jnp.int32)
    col_p = jnp.concatenate([col, jnp.full((EPAD - E,), N, jnp.int32)])
    ea_t = jnp.concatenate(
        [edge_attr.T, jnp.zeros((EDGE_DIM, EPAD - E), jnp.float32)], axis=1)

    a = _compute_a(x, W_node, att2)
    a_t = jnp.concatenate(
        [a, jnp.zeros((NPAD - N, HEADS), jnp.float32)], axis=0).T
    b_t = _compute_b(ea_t, W_edge, att2)

    tables = _sc_tables(col_p, b_t.reshape(-1), ea_t.reshape(-1),
                        a_t.reshape(-1))
    tables = tables.reshape(32, NPAD, TW)

    return _compute_out(x, tables, W_node, W_edge, W_out,
                        b_out.reshape(1, OUT_CH))


# speedup vs baseline: 2.5587x; 2.5587x over previous
import jax
import jax.numpy as jnp
from jax import lax
from jax.experimental import pallas as pl
from jax.experimental.pallas import tpu as pltpu
from jax.experimental.pallas import tpu_sc as plsc

N = 10000
E = 160000
IN_CH = 256
OUT_CH = 256
HEADS = 4
EDGE_DIM = 4

NPAD = 10112
TW = 24
AW = 5
NWORKERS = 32
SCHUNK = 512
EPT = 5120
EPAD = EPT * NWORKERS


def _a_body(x_ref, wn_ref, we_ref, att_ref, a_ref, w_ref):
    cols = []
    for h in range(HEADS):
        vh = jnp.sum(wn_ref[:, h * OUT_CH:(h + 1) * OUT_CH] * att_ref[h, :][None, :],
                     axis=1)
        cols.append(vh[:, None])
    v = jnp.concatenate(cols, axis=1)
    a_ref[...] = jnp.dot(x_ref[...], v, preferred_element_type=jnp.float32)
    rows = []
    for d in range(EDGE_DIM):
        ents = []
        for h in range(HEADS):
            ents.append(jnp.sum(we_ref[d:d + 1, h * OUT_CH:(h + 1) * OUT_CH]
                                * att_ref[h:h + 1, :], axis=1, keepdims=True))
        rows.append(jnp.concatenate(ents, axis=1))
    w_ref[...] = jnp.concatenate(rows, axis=0)


def _compute_a(x, w_node, w_edge, att2):
    blk = 2000
    return pl.pallas_call(
        _a_body,
        grid=(N // blk,),
        in_specs=[
            pl.BlockSpec((blk, IN_CH), lambda i: (i, 0)),
            pl.BlockSpec((IN_CH, OUT_CH * HEADS), lambda i: (0, 0)),
            pl.BlockSpec((EDGE_DIM, OUT_CH * HEADS), lambda i: (0, 0)),
            pl.BlockSpec((HEADS, OUT_CH), lambda i: (0, 0)),
        ],
        out_specs=[
            pl.BlockSpec((blk, HEADS), lambda i: (i, 0)),
            pl.BlockSpec((EDGE_DIM, HEADS), lambda i: (0, 0)),
        ],
        out_shape=[
            jax.ShapeDtypeStruct((N, HEADS), jnp.float32),
            jax.ShapeDtypeStruct((EDGE_DIM, HEADS), jnp.float32),
        ],
    )(x, w_node, w_edge, att2)


def _sc_body(col_hbm, ea_hbm, a_hbm, w_hbm, out_hbm,
             a_v, col_v, w_v, ea_v, rows_v, table_sh, sem_in, sem_s0, sem_s1):
    cid = lax.axis_index("c")
    sid = lax.axis_index("s")
    wid = sid * 2 + cid

    in_descs = [
        pltpu.async_copy(col_hbm.at[pl.ds(wid * (EPT // 128), EPT // 128)],
                         col_v, sem_in),
        pltpu.async_copy(a_hbm, a_v, sem_in),
        pltpu.async_copy(w_hbm, w_v, sem_in),
    ]
    for d in range(EDGE_DIM):
        in_descs.append(pltpu.async_copy(
            ea_hbm.at[pl.ds(d * EPAD + wid * EPT, EPT)], ea_v.at[d], sem_in))

    zero16 = jnp.zeros((16,), jnp.float32)

    @pl.loop(0, 2 * SCHUNK)
    def _zero_rows(i):
        rows_v[i, pl.ds(0, 16)] = zero16
        rows_v[i, pl.ds(TW - 16, 16)] = zero16

    share = NPAD // 16
    pltpu.sync_copy(rows_v.at[pl.ds(0, share)],
                    table_sh.at[pl.ds(sid * share, share)])
    plsc.subcore_barrier()

    for dsc in in_descs:
        dsc.wait()

    wvec = w_v[pl.ds(0, 16)]
    w_s = [[wvec[d * HEADS + h] for h in range(HEADS)]
           for d in range(EDGE_DIM)]

    iota16 = lax.iota(jnp.int32, 16)
    sem_s = [sem_s0, sem_s1]
    out_descs = [None, None]

    for ci in range(EPT // SCHUNK):
        buf = ci % 2
        if out_descs[buf] is not None:
            for dsc in out_descs[buf]:
                dsc.wait()
        rbase = buf * SCHUNK

        @pl.loop(0, SCHUNK // 128)
        def _compute(j, ci=ci, rbase=rbase):
            r = ci * (SCHUNK // 128) + j

            @pl.loop(0, 8)
            def _group(k):
                col16 = col_v[r, pl.ds(k * 16, 16)]
                e0 = r * 128 + k * 16
                l16 = rbase + (j * 8 + k) * 16 + iota16
                cola = col16 * AW
                ea = [ea_v[d, pl.ds(e0, 16)] for d in range(EDGE_DIM)]
                for h in range(HEADS):
                    hv = jnp.full((16,), h, jnp.int32)
                    ah = plsc.load_gather(a_v, [cola + h])
                    bh = ((ea[0] * w_s[0][h] + ea[1] * w_s[1][h])
                          + (ea[2] * w_s[2][h] + ea[3] * w_s[3][h]))
                    t = ah + bh
                    t = jnp.maximum(t, 0.2 * t)
                    p = jnp.exp(t)
                    plsc.store_scatter(rows_v, [l16, hv], p)
                    for d in range(EDGE_DIM):
                        cv = jnp.full((16,), HEADS + h * EDGE_DIM + d,
                                      jnp.int32)
                        plsc.store_scatter(rows_v, [l16, cv], p * ea[d])

        dsl = []
        for j in range(SCHUNK // 128):
            dsl.append(pltpu.async_copy(
                rows_v.at[pl.ds(rbase + j * 128, 128)],
                table_sh.at[col_v.at[ci * (SCHUNK // 128) + j]],
                sem_s[buf], add=True))
        out_descs[buf] = dsl

    for buf in range(2):
        for dsc in out_descs[buf]:
            dsc.wait()

    plsc.subcore_barrier()
    pltpu.sync_copy(table_sh.at[pl.ds(sid * share, share)],
                    out_hbm.at[cid, pl.ds(sid * share, share)])


def _sc_tables(col2d, ea_f, a_f, w_f):
    mesh = plsc.VectorSubcoreMesh(core_axis_name="c", subcore_axis_name="s",
                                  num_cores=2, num_subcores=16)
    fn = pl.kernel(
        _sc_body,
        out_type=jax.ShapeDtypeStruct((2, NPAD, TW), jnp.float32),
        mesh=mesh,
        compiler_params=pltpu.CompilerParams(needs_layout_passes=False,
                                             use_tc_tiling_on_sc=False),
        scratch_types=[
            pltpu.VMEM((NPAD * AW,), jnp.float32),
            pltpu.VMEM((EPT // 128, 128), jnp.int32),
            pltpu.VMEM((16,), jnp.float32),
            pltpu.VMEM((EDGE_DIM, EPT), jnp.float32),
            pltpu.VMEM((2 * SCHUNK, TW), jnp.float32),
            pltpu.VMEM_SHARED((NPAD, TW), jnp.float32),
            pltpu.SemaphoreType.DMA,
            pltpu.SemaphoreType.DMA,
            pltpu.SemaphoreType.DMA,
        ],
    )
    return fn(col2d, ea_f, a_f, w_f)


def _out_body(x_ref, t0_ref, t1_ref, wn_ref, we_ref, wo_ref, bo_ref, o_ref):
    t = t0_ref[...] + t1_ref[...]
    s = t[:, 0:HEADS]
    inv = 1.0 / (s + 1e-8)
    x = x_ref[...]
    acc = None
    for h in range(HEADS):
        xh = jnp.dot(x, wn_ref[:, h * OUT_CH:(h + 1) * OUT_CH],
                     preferred_element_type=jnp.float32)
        sh = s[:, h:h + 1] * inv[:, h:h + 1]
        gh = t[:, HEADS + EDGE_DIM * h:HEADS + EDGE_DIM * (h + 1)] * inv[:, h:h + 1]
        zh = jnp.dot(gh, we_ref[:, h * OUT_CH:(h + 1) * OUT_CH],
                     preferred_element_type=jnp.float32)
        term = xh * sh + zh
        contrib = jnp.dot(term, wo_ref[h * OUT_CH:(h + 1) * OUT_CH, :],
                          preferred_element_type=jnp.float32)
        acc = contrib if acc is None else acc + contrib
    o_ref[...] = acc + bo_ref[...]


def _compute_out(x, t0, t1, w_node, w_edge, w_out, b_out2):
    blk = 400
    return pl.pallas_call(
        _out_body,
        grid=(N // blk,),
        in_specs=[
            pl.BlockSpec((blk, IN_CH), lambda i: (i, 0)),
            pl.BlockSpec((blk, TW), lambda i: (i, 0)),
            pl.BlockSpec((blk, TW), lambda i: (i, 0)),
            pl.BlockSpec((IN_CH, OUT_CH * HEADS), lambda i: (0, 0)),
            pl.BlockSpec((EDGE_DIM, OUT_CH * HEADS), lambda i: (0, 0)),
            pl.BlockSpec((OUT_CH * HEADS, OUT_CH), lambda i: (0, 0)),
            pl.BlockSpec((1, OUT_CH), lambda i: (0, 0)),
        ],
        out_specs=pl.BlockSpec((blk, OUT_CH), lambda i: (i, 0)),
        out_shape=jax.ShapeDtypeStruct((N, OUT_CH), jnp.float32),
    )(x, t0, t1, w_node, w_edge, w_out, b_out2)


def kernel(x, edge_index, edge_attr, W_node, W_edge, att, W_out, b_out):
    att2 = att.reshape(HEADS, OUT_CH)
    col = edge_index[1].astype(jnp.int32)
    col_p = jnp.concatenate(
        [col, jnp.full((EPAD - E,), N, jnp.int32)]).reshape(EPAD // 128, 128)
    ea_t = jnp.concatenate(
        [edge_attr.T, jnp.zeros((EDGE_DIM, EPAD - E), jnp.float32)], axis=1)

    a, w_att = _compute_a(x, W_node, W_edge, att2)
    a_pad = jnp.zeros((NPAD, AW), jnp.float32).at[:N, :HEADS].set(a)

    tables = _sc_tables(col_p, ea_t.reshape(-1), a_pad.reshape(-1),
                        w_att.reshape(-1))

    return _compute_out(x, tables[0], tables[1], W_node, W_edge, W_out,
                        b_out.reshape(1, OUT_CH))


# speedup vs baseline: 2.6314x; 1.0284x over previous
import jax
import jax.numpy as jnp
from jax import lax
from jax.experimental import pallas as pl
from jax.experimental.pallas import tpu as pltpu
from jax.experimental.pallas import tpu_sc as plsc

N = 10000
E = 160000
IN_CH = 256
OUT_CH = 256
HEADS = 4
EDGE_DIM = 4

NPAD = 10112
TW = 24
AW = 5
NWORKERS = 32
SCHUNK = 512
EPT = 5120
EPAD = EPT * NWORKERS


def _a_body(x_ref, wn_ref, we_ref, wo_ref, att_ref, a_ref, w_ref, k_ref,
            ww_ref):
    cols = []
    for h in range(HEADS):
        vh = jnp.sum(wn_ref[:, h * OUT_CH:(h + 1) * OUT_CH] * att_ref[h, :][None, :],
                     axis=1)
        cols.append(vh[:, None])
    v = jnp.concatenate(cols, axis=1)
    a_ref[...] = jnp.dot(x_ref[...], v, preferred_element_type=jnp.float32)
    rows = []
    for d in range(EDGE_DIM):
        ents = []
        for h in range(HEADS):
            ents.append(jnp.sum(we_ref[d:d + 1, h * OUT_CH:(h + 1) * OUT_CH]
                                * att_ref[h:h + 1, :], axis=1, keepdims=True))
        rows.append(jnp.concatenate(ents, axis=1))
    w_ref[...] = jnp.concatenate(rows, axis=0)
    for h in range(HEADS):
        wo_h = wo_ref[h * OUT_CH:(h + 1) * OUT_CH, :]
        k_ref[:, h * OUT_CH:(h + 1) * OUT_CH] = jnp.dot(
            wn_ref[:, h * OUT_CH:(h + 1) * OUT_CH], wo_h,
            preferred_element_type=jnp.float32)
        ww_ref[h * EDGE_DIM:(h + 1) * EDGE_DIM, :] = jnp.dot(
            we_ref[:, h * OUT_CH:(h + 1) * OUT_CH], wo_h,
            preferred_element_type=jnp.float32)


def _compute_a(x, w_node, w_edge, w_out, att2):
    blk = 2000
    return pl.pallas_call(
        _a_body,
        grid=(N // blk,),
        in_specs=[
            pl.BlockSpec((blk, IN_CH), lambda i: (i, 0)),
            pl.BlockSpec((IN_CH, OUT_CH * HEADS), lambda i: (0, 0)),
            pl.BlockSpec((EDGE_DIM, OUT_CH * HEADS), lambda i: (0, 0)),
            pl.BlockSpec((OUT_CH * HEADS, OUT_CH), lambda i: (0, 0)),
            pl.BlockSpec((HEADS, OUT_CH), lambda i: (0, 0)),
        ],
        out_specs=[
            pl.BlockSpec((blk, HEADS), lambda i: (i, 0)),
            pl.BlockSpec((EDGE_DIM, HEADS), lambda i: (0, 0)),
            pl.BlockSpec((IN_CH, OUT_CH * HEADS), lambda i: (0, 0)),
            pl.BlockSpec((HEADS * EDGE_DIM, OUT_CH), lambda i: (0, 0)),
        ],
        out_shape=[
            jax.ShapeDtypeStruct((N, HEADS), jnp.float32),
            jax.ShapeDtypeStruct((EDGE_DIM, HEADS), jnp.float32),
            jax.ShapeDtypeStruct((IN_CH, OUT_CH * HEADS), jnp.float32),
            jax.ShapeDtypeStruct((HEADS * EDGE_DIM, OUT_CH), jnp.float32),
        ],
    )(x, w_node, w_edge, w_out, att2)


def _sc_body(col_hbm, ea_hbm, a_hbm, w_hbm, out_hbm,
             a_v, col_v, w_v, ea_v, rows_v, table_sh, sem_in, sem_s0, sem_s1):
    cid = lax.axis_index("c")
    sid = lax.axis_index("s")
    wid = sid * 2 + cid

    in_descs = [
        pltpu.async_copy(col_hbm.at[pl.ds(wid * (EPT // 128), EPT // 128)],
                         col_v, sem_in),
        pltpu.async_copy(a_hbm, a_v, sem_in),
        pltpu.async_copy(w_hbm, w_v, sem_in),
    ]
    for d in range(EDGE_DIM):
        in_descs.append(pltpu.async_copy(
            ea_hbm.at[pl.ds(d * EPAD + wid * EPT, EPT)], ea_v.at[d], sem_in))

    zero16 = jnp.zeros((16,), jnp.float32)

    @pl.loop(0, 2 * SCHUNK)
    def _zero_rows(i):
        rows_v[i, pl.ds(0, 16)] = zero16
        rows_v[i, pl.ds(TW - 16, 16)] = zero16

    share = NPAD // 16
    pltpu.sync_copy(rows_v.at[pl.ds(0, share)],
                    table_sh.at[pl.ds(sid * share, share)])
    plsc.subcore_barrier()

    for dsc in in_descs:
        dsc.wait()

    wvec = w_v[pl.ds(0, 16)]
    w_s = [[wvec[d * HEADS + h] for h in range(HEADS)]
           for d in range(EDGE_DIM)]

    iota16 = lax.iota(jnp.int32, 16)
    sem_s = [sem_s0, sem_s1]
    out_descs = [None, None]

    for ci in range(EPT // SCHUNK):
        buf = ci % 2
        if out_descs[buf] is not None:
            for dsc in out_descs[buf]:
                dsc.wait()
        rbase = buf * SCHUNK

        @pl.loop(0, SCHUNK // 128)
        def _compute(j, ci=ci, rbase=rbase):
            r = ci * (SCHUNK // 128) + j

            @pl.loop(0, 8)
            def _group(k):
                col16 = col_v[r, pl.ds(k * 16, 16)]
                e0 = r * 128 + k * 16
                l16 = rbase + (j * 8 + k) * 16 + iota16
                cola = col16 * AW
                ea = [ea_v[d, pl.ds(e0, 16)] for d in range(EDGE_DIM)]
                for h in range(HEADS):
                    hv = jnp.full((16,), h, jnp.int32)
                    ah = plsc.load_gather(a_v, [cola + h])
                    bh = ((ea[0] * w_s[0][h] + ea[1] * w_s[1][h])
                          + (ea[2] * w_s[2][h] + ea[3] * w_s[3][h]))
                    t = ah + bh
                    t = jnp.maximum(t, 0.2 * t)
                    p = jnp.exp(t)
                    plsc.store_scatter(rows_v, [l16, hv], p)
                    for d in range(EDGE_DIM):
                        cv = jnp.full((16,), HEADS + h * EDGE_DIM + d,
                                      jnp.int32)
                        plsc.store_scatter(rows_v, [l16, cv], p * ea[d])

        dsl = []
        for j in range(SCHUNK // 128):
            dsl.append(pltpu.async_copy(
                rows_v.at[pl.ds(rbase + j * 128, 128)],
                table_sh.at[col_v.at[ci * (SCHUNK // 128) + j]],
                sem_s[buf], add=True))
        out_descs[buf] = dsl

    for buf in range(2):
        for dsc in out_descs[buf]:
            dsc.wait()

    plsc.subcore_barrier()
    pltpu.sync_copy(table_sh.at[pl.ds(sid * share, share)],
                    out_hbm.at[cid, pl.ds(sid * share, share)])


def _sc_tables(col2d, ea_f, a_f, w_f):
    mesh = plsc.VectorSubcoreMesh(core_axis_name="c", subcore_axis_name="s",
                                  num_cores=2, num_subcores=16)
    fn = pl.kernel(
        _sc_body,
        out_type=jax.ShapeDtypeStruct((2, NPAD, TW), jnp.float32),
        mesh=mesh,
        compiler_params=pltpu.CompilerParams(needs_layout_passes=False,
                                             use_tc_tiling_on_sc=False),
        scratch_types=[
            pltpu.VMEM((NPAD * AW,), jnp.float32),
            pltpu.VMEM((EPT // 128, 128), jnp.int32),
            pltpu.VMEM((16,), jnp.float32),
            pltpu.VMEM((EDGE_DIM, EPT), jnp.float32),
            pltpu.VMEM((2 * SCHUNK, TW), jnp.float32),
            pltpu.VMEM_SHARED((NPAD, TW), jnp.float32),
            pltpu.SemaphoreType.DMA,
            pltpu.SemaphoreType.DMA,
            pltpu.SemaphoreType.DMA,
        ],
    )
    return fn(col2d, ea_f, a_f, w_f)


def _out_body(x_ref, t0_ref, t1_ref, k_ref, ww_ref, bo_ref, o_ref):
    t = t0_ref[...] + t1_ref[...]
    s = t[:, 0:HEADS]
    inv = 1.0 / (s + 1e-8)
    x = x_ref[...]
    gcols = []
    acc = None
    for h in range(HEADS):
        yh = jnp.dot(x, k_ref[:, h * OUT_CH:(h + 1) * OUT_CH],
                     preferred_element_type=jnp.float32)
        sh = s[:, h:h + 1] * inv[:, h:h + 1]
        contrib = yh * sh
        acc = contrib if acc is None else acc + contrib
        gcols.append(t[:, HEADS + EDGE_DIM * h:HEADS + EDGE_DIM * (h + 1)]
                     * inv[:, h:h + 1])
    g_flat = jnp.concatenate(gcols, axis=1)
    z = jnp.dot(g_flat, ww_ref[...], preferred_element_type=jnp.float32)
    o_ref[...] = acc + z + bo_ref[...]


def _compute_out(x, t0, t1, k_mat, wewo, b_out2):
    blk = 400
    return pl.pallas_call(
        _out_body,
        grid=(N // blk,),
        in_specs=[
            pl.BlockSpec((blk, IN_CH), lambda i: (i, 0)),
            pl.BlockSpec((blk, TW), lambda i: (i, 0)),
            pl.BlockSpec((blk, TW), lambda i: (i, 0)),
            pl.BlockSpec((IN_CH, OUT_CH * HEADS), lambda i: (0, 0)),
            pl.BlockSpec((HEADS * EDGE_DIM, OUT_CH), lambda i: (0, 0)),
            pl.BlockSpec((1, OUT_CH), lambda i: (0, 0)),
        ],
        out_specs=pl.BlockSpec((blk, OUT_CH), lambda i: (i, 0)),
        out_shape=jax.ShapeDtypeStruct((N, OUT_CH), jnp.float32),
    )(x, t0, t1, k_mat, wewo, b_out2)


def kernel(x, edge_index, edge_attr, W_node, W_edge, att, W_out, b_out):
    att2 = att.reshape(HEADS, OUT_CH)
    col = edge_index[1].astype(jnp.int32)
    col_p = jnp.concatenate(
        [col, jnp.full((EPAD - E,), N, jnp.int32)]).reshape(EPAD // 128, 128)
    ea_t = jnp.concatenate(
        [edge_attr.T, jnp.zeros((EDGE_DIM, EPAD - E), jnp.float32)], axis=1)

    a, w_att, k_mat, wewo = _compute_a(x, W_node, W_edge, W_out, att2)
    a_pad = jnp.zeros((NPAD, AW), jnp.float32).at[:N, :HEADS].set(a)

    tables = _sc_tables(col_p, ea_t.reshape(-1), a_pad.reshape(-1),
                        w_att.reshape(-1))

    return _compute_out(x, tables[0], tables[1], k_mat, wewo,
                        b_out.reshape(1, OUT_CH))
